# Initial kernel scaffold; baseline (speedup 1.0000x reference)
import jax
import jax.numpy as jnp
from jax.experimental import pallas as pl


def kernel(human_bt_n3, object_bt_m3, s_h_bt_n, s_o_bt_m, W1, b1, W2, b2):
    raise NotImplementedError("write your pallas kernel here")


# speedup vs baseline: 1.0939x; 1.0939x over previous
import functools

import jax
import jax.numpy as jnp
from jax.experimental import pallas as pl
from jax.experimental.pallas import tpu as pltpu

TAU = 0.05


def _encoder_kernel(ht_ref, o_ref, sh_ref, w1_ref, b1_ref, w2_ref, b2_ref,
                    out_ref, *, nh, no, kqs):
    f32 = jnp.float32
    h3 = ht_ref[0]
    hx = h3[0:1, :]
    hy = h3[1:2, :]
    hz = h3[2:3, :]
    o3 = o_ref[0]
    ox = o3[:, 0:1]
    oy = o3[:, 1:2]
    oz = o3[:, 2:3]

    rp = lambda x: x.astype(jnp.bfloat16).astype(f32)
    a2 = hx * hx + hy * hy + hz * hz
    b2 = ox * ox + oy * oy + oz * oz
    cross = rp(ox) * rp(hx) + rp(oy) * rp(hy) + rp(oz) * rp(hz)
    sq = (a2 + b2) - 2.0 * cross
    sqc = jnp.maximum(sq, 1e-12)

    min_sq_h = jnp.min(sqc, axis=0, keepdims=True)
    dmin_h = jnp.sqrt(min_sq_h)
    min_sq_o = jnp.min(sqc, axis=1, keepdims=True)
    dmin_o = jnp.sqrt(min_sq_o)

    iota_m = jax.lax.broadcasted_iota(jnp.int32, (no, nh), 0)
    eligible = sqc == min_sq_h
    first = jnp.min(jnp.where(eligible, iota_m, no), axis=0, keepdims=True)
    maskf = (iota_m == first).astype(f32)
    onx = jnp.sum(maskf * ox, axis=0, keepdims=True)
    ony = jnp.sum(maskf * oy, axis=0, keepdims=True)
    onz = jnp.sum(maskf * oz, axis=0, keepdims=True)
    vx = onx - hx
    vy = ony - hy
    vz = onz - hz
    nrm = jnp.sqrt(jnp.maximum(vx * vx + vy * vy + vz * vz, 1e-6))
    dirx = vx / nrm
    diry = vy / nrm
    dirz = vz / nrm

    sh = sh_ref[0]
    w_h = jnp.exp(-dmin_h * (1.0 / TAU)) * sh

    ii = jax.lax.broadcasted_iota(jnp.int32, (nh, nh), 0)
    jj = jax.lax.broadcasted_iota(jnp.int32, (nh, nh), 1)
    kcol = jnp.sum(jnp.where(ii == jj, min_sq_h, 0.0), axis=1,
                   keepdims=True)
    lt = (kcol < min_sq_h).astype(f32)
    tie = ((kcol == min_sq_h) & (ii < jj)).astype(f32)
    rank = jnp.sum(lt + tie, axis=0, keepdims=True)

    inv_nh = 1.0 / nh
    f1 = jnp.sum(dmin_h, keepdims=True) * inv_nh
    f2 = jnp.min(dmin_h, keepdims=True)
    q = []
    for kq in kqs:
        sel = (rank < float(kq)).astype(f32)
        q.append(jnp.sum(dmin_h * sel, keepdims=True) * (1.0 / kq))
    f6 = jnp.sum(w_h, keepdims=True) * inv_nh
    f7 = jnp.sum(dirx, keepdims=True) * inv_nh
    f8 = jnp.sum(diry, keepdims=True) * inv_nh
    f9 = jnp.sum(dirz, keepdims=True) * inv_nh
    f10 = jnp.sum(dmin_o, keepdims=True) * (1.0 / no)

    feats = (f1, f2, q[0], q[1], q[2], f6, f7, f8, f9, f10)
    w1 = rp(w1_ref[:])
    acc = b1_ref[:]
    for k, f in enumerate(feats):
        acc = acc + rp(f) * w1[k:k + 1, :]
    hid = jnp.maximum(acc, 0.0)
    out = jnp.dot(rp(hid), rp(w2_ref[:]),
                  preferred_element_type=f32) + b2_ref[:]
    out_ref[0] = out


def kernel(human_bt_n3, object_bt_m3, s_h_bt_n, s_o_bt_m, W1, b1, W2, b2):
    B, T, Nh, _ = human_bt_n3.shape
    No = object_bt_m3.shape[2]
    BT = B * T
    Dout = W2.shape[1]
    ht = human_bt_n3.reshape(BT, Nh, 3).transpose(0, 2, 1)
    o = object_bt_m3.reshape(BT, No, 3)
    sh = s_h_bt_n.reshape(BT, 1, Nh)
    b1r = b1.reshape(1, -1)
    b2r = b2.reshape(1, -1)
    kqs = tuple(int(max(1, round(qv * Nh))) for qv in (0.2, 0.5, 0.8))

    body = functools.partial(_encoder_kernel, nh=Nh, no=No, kqs=kqs)
    out = pl.pallas_call(
        body,
        grid=(BT,),
        in_specs=[
            pl.BlockSpec((1, 3, Nh), lambda i: (i, 0, 0)),
            pl.BlockSpec((1, No, 3), lambda i: (i, 0, 0)),
            pl.BlockSpec((1, 1, Nh), lambda i: (i, 0, 0)),
            pl.BlockSpec(W1.shape, lambda i: (0, 0)),
            pl.BlockSpec(b1r.shape, lambda i: (0, 0)),
            pl.BlockSpec(W2.shape, lambda i: (0, 0)),
            pl.BlockSpec(b2r.shape, lambda i: (0, 0)),
        ],
        out_specs=pl.BlockSpec((1, 1, Dout), lambda i: (i, 0, 0)),
        out_shape=jax.ShapeDtypeStruct((BT, 1, Dout), jnp.float32),
        compiler_params=pltpu.CompilerParams(
            dimension_semantics=("parallel",)),
    )(ht, o, sh, W1, b1r, W2, b2r)
    return out.reshape(B, T, Dout)
